# Initial kernel scaffold; baseline (speedup 1.0000x reference)
import jax
import jax.numpy as jnp
from jax.experimental import pallas as pl


def kernel(edge_index_0, edge_index_1, edge_index_2, liked_indices, unknown_indices, disliked_indices, embed, W1_0, b1_0, W1_1, b1_1, W1_2, b1_2, W2_0, b2_0, W2_1, b2_1, W2_2, b2_2, Wu, bu):
    raise NotImplementedError("write your pallas kernel here")


# speedup vs baseline: 3.7963x; 3.7963x over previous
import functools

import jax
import jax.numpy as jnp
from jax import lax
from jax.experimental import pallas as pl
from jax.experimental.pallas import tpu as pltpu
from jax.experimental.pallas import tpu_sc as plsc

N_NODES = 50000
E_PER = 266667
IN_SIZE, HIDDEN, EMB = 128, 64, 64
BATCH, HIST = 256, 20

NC, NS = 2, 16
CHUNK = 128
N_CH = 131
EW = N_CH * CHUNK
E_PAD = NS * EW
NPAD = 50176
TPT = NPAD // NS
STG = TPT // 4
QW = 16
NQ = 4
HB = 1024

_mesh = functools.partial(
    plsc.VectorSubcoreMesh, core_axis_name="c", subcore_axis_name="s",
    num_cores=NC, num_subcores=NS)

_sc_params = functools.partial(
    pltpu.CompilerParams, use_tc_tiling_on_sc=False)


def _f32(*shape):
    return jax.ShapeDtypeStruct(shape, jnp.float32)


def _k1_body(x_ref, w0, w1, w2, b0, b1, b2, *outs):
    x = x_ref[...]
    for e, (w, b) in enumerate(((w0, b0), (w1, b1), (w2, b2))):
        y = jnp.dot(x, w[...], preferred_element_type=jnp.float32) + b[...]
        for q in range(NQ):
            outs[e * NQ + q][...] = y[:, q * QW:(q + 1) * QW]


def _linear1(embed, ws, bs):
    grid = NPAD // HB
    qspec = pl.BlockSpec((HB, QW), lambda i: (i, 0))
    return pl.pallas_call(
        _k1_body,
        grid=(grid,),
        in_specs=[pl.BlockSpec((HB, IN_SIZE), lambda i: (i, 0))]
        + [pl.BlockSpec((IN_SIZE, HIDDEN), lambda i: (0, 0))] * 3
        + [pl.BlockSpec((1, HIDDEN), lambda i: (0, 0))] * 3,
        out_specs=[qspec] * (3 * NQ),
        out_shape=[_f32(N_NODES, QW)] * (3 * NQ),
    )(embed, *ws, *bs)


def _make_agg(want_cnt):
    outs = [_f32(NQ * NPAD, QW)] * 3
    if want_cnt:
        outs.append(_f32(3 * NPAD))

    scratch = [
        pltpu.VMEM((EW,), jnp.int32),
        pltpu.VMEM((EW,), jnp.int32),
        pltpu.VMEM((CHUNK,), jnp.int32),
        pltpu.VMEM((CHUNK,), jnp.int32),
        pltpu.VMEM((CHUNK, QW), jnp.float32),
        pltpu.VMEM((CHUNK,), jnp.float32),
        pltpu.VMEM((STG, QW), jnp.float32),
        pltpu.VMEM((STG, QW), jnp.float32),
        pltpu.VMEM((TPT,), jnp.float32),
        pltpu.VMEM((TPT,), jnp.float32),
        pltpu.VMEM_SHARED((NPAD, QW), jnp.float32),
        pltpu.VMEM_SHARED((NPAD,), jnp.float32),
    ]

    def body(s0, d0, s1, d1, s2, d2, *rest):
        tbls = rest[:3 * NQ]
        rest = rest[3 * NQ:]
        zrows, zcnt = rest[:2]
        rest = rest[2:]
        if want_cnt:
            o0, o1, o2, ocnt = rest[:4]
            rest = rest[4:]
        else:
            o0, o1, o2 = rest[:3]
            rest = rest[3:]
        (srcb, dstb, idxv, dstv, rowsv, onesv, stg, wbuf, cbuf, cbuf2,
         acc, cntacc) = rest
        c = lax.axis_index("c")
        s = lax.axis_index("s")
        r0 = s * TPT

        if want_cnt:
            for j in range(CHUNK // 16):
                onesv[pl.ds(j * 16, 16)] = jnp.full((16,), 1.0, jnp.float32)

        srcs = (s0, s1, s2)
        dsts = (d0, d1, d2)
        souts = (o0, o1, o2)

        pltpu.sync_copy(zrows, stg)
        if want_cnt:
            pltpu.sync_copy(zcnt, cbuf)

        for e in range(3):
            pltpu.sync_copy(srcs[e].at[pl.ds(s * EW, EW)], srcb)
            pltpu.sync_copy(dsts[e].at[pl.ds(s * EW, EW)], dstb)

            for p in range(2):
                do_cnt = want_cnt and p == 0
                for k in range(TPT // STG):
                    pltpu.sync_copy(stg, acc.at[pl.ds(r0 + k * STG, STG)])
                if do_cnt:
                    @pl.when(c == 0)
                    def _():
                        pltpu.sync_copy(cbuf, cntacc.at[pl.ds(r0, TPT)])
                plsc.subcore_barrier()

                def chunk(k, _, e=e, p=p, do_cnt=do_cnt):
                    kb = k * CHUNK
                    for j in range(CHUNK // 16):
                        idxv[pl.ds(j * 16, 16)] = srcb[pl.ds(kb + j * 16, 16)]
                        dstv[pl.ds(j * 16, 16)] = dstb[pl.ds(kb + j * 16, 16)]

                    @pl.when(c == 0)
                    def _():
                        pltpu.sync_copy(tbls[e * NQ + p].at[idxv], rowsv)

                    @pl.when(c == 1)
                    def _():
                        pltpu.sync_copy(tbls[e * NQ + 2 + p].at[idxv], rowsv)

                    pltpu.sync_copy(rowsv, acc.at[dstv], add=True)
                    if do_cnt:
                        @pl.when(c == 0)
                        def _():
                            pltpu.sync_copy(onesv, cntacc.at[dstv], add=True)
                    return 0

                lax.fori_loop(0, N_CH, chunk, 0)
                plsc.subcore_barrier()

                q_off = (2 * c + p) * NPAD
                for k in range(TPT // STG):
                    pltpu.sync_copy(acc.at[pl.ds(r0 + k * STG, STG)], wbuf)
                    pltpu.sync_copy(
                        wbuf, souts[e].at[pl.ds(q_off + r0 + k * STG, STG)])
                if do_cnt:
                    @pl.when(c == 0)
                    def _(e=e):
                        pltpu.sync_copy(cntacc.at[pl.ds(r0, TPT)], cbuf2)
                        pltpu.sync_copy(cbuf2,
                                        ocnt.at[pl.ds(e * NPAD + r0, TPT)])
                plsc.subcore_barrier()

    return pl.kernel(body, out_type=outs, mesh=_mesh(),
                     scratch_types=scratch,
                     compiler_params=_sc_params())


def _k2_body(*refs):
    qs = refs[:3 * NQ]
    cnt_ref = refs[3 * NQ]
    w0, w1, w2, b0, b1, b2 = refs[3 * NQ + 1:3 * NQ + 7]
    outs = refs[3 * NQ + 7:]
    cnt = cnt_ref[...]
    h = jnp.zeros((HB, HIDDEN), jnp.float32)
    for e in range(3):
        he = jnp.concatenate([qs[e * NQ + q][...] for q in range(NQ)], axis=1)
        inv = 1.0 / jnp.maximum(cnt[e], 1.0)
        h = h + he * inv[:, None]
    h = jnp.where(h >= 0, h, 0.01 * h)
    for e, (w, b) in enumerate(((w0, b0), (w1, b1), (w2, b2))):
        y = jnp.dot(h, w[...], preferred_element_type=jnp.float32) + b[...]
        for q in range(NQ):
            outs[e * NQ + q][...] = y[:, q * QW:(q + 1) * QW]


def _quarter_specs():
    specs = []
    for _e in range(3):
        for q in range(NQ):
            specs.append(
                pl.BlockSpec((HB, QW),
                             lambda i, q=q: (q * (NPAD // HB) + i, 0)))
    return specs


def _mid(s1s, cnt, ws, bs):
    grid = NPAD // HB
    in_specs = _quarter_specs()
    ops = [s1s[e] for e in range(3) for _q in range(NQ)]
    in_specs.append(pl.BlockSpec((3, HB), lambda i: (0, i)))
    ops.append(cnt)
    in_specs += [pl.BlockSpec((HIDDEN, HIDDEN), lambda i: (0, 0))] * 3
    in_specs += [pl.BlockSpec((1, HIDDEN), lambda i: (0, 0))] * 3
    ops += list(ws) + list(bs)
    qspec = pl.BlockSpec((HB, QW), lambda i: (i, 0))
    return pl.pallas_call(
        _k2_body, grid=(grid,), in_specs=in_specs,
        out_specs=[qspec] * (3 * NQ), out_shape=[_f32(N_NODES, QW)] * (3 * NQ),
    )(*ops)


def _k3_body(*refs):
    qs = refs[:3 * NQ]
    cnt_ref = refs[3 * NQ]
    out_ref = refs[3 * NQ + 1]
    cnt = cnt_ref[...]
    h = jnp.zeros((HB, EMB), jnp.float32)
    for e in range(3):
        he = jnp.concatenate([qs[e * NQ + q][...] for q in range(NQ)], axis=1)
        inv = 1.0 / jnp.maximum(cnt[e], 1.0)
        h = h + he * inv[:, None]
    out_ref[...] = jax.nn.sigmoid(h)


def _entities(s2s, cnt):
    grid = NPAD // HB
    in_specs = _quarter_specs()
    ops = [s2s[e] for e in range(3) for _q in range(NQ)]
    in_specs.append(pl.BlockSpec((3, HB), lambda i: (0, i)))
    ops.append(cnt)
    return pl.pallas_call(
        _k3_body, grid=(grid,), in_specs=in_specs,
        out_specs=pl.BlockSpec((HB, EMB), lambda i: (i, 0)),
        out_shape=_f32(N_NODES, EMB),
    )(*ops)


N_IDX = 3 * BATCH * HIST
IPW = N_IDX // (NC * NS)
GC = 96


def _gather_body(ent_h, idx_h, out_h, idxb, idxv, rowsv):
    c = lax.axis_index("c")
    s = lax.axis_index("s")
    base = (s * NC + c) * IPW
    pltpu.sync_copy(idx_h.at[pl.ds(base, IPW)], idxb)
    for k in range(IPW // GC):
        for j in range(GC // 16):
            idxv[pl.ds(j * 16, 16)] = idxb[pl.ds(k * GC + j * 16, 16)]
        pltpu.sync_copy(ent_h.at[idxv], rowsv)
        pltpu.sync_copy(rowsv, out_h.at[pl.ds(base + k * GC, GC)])


_gather_hist = pl.kernel(
    _gather_body, out_type=_f32(N_IDX, EMB), mesh=_mesh(),
    scratch_types=[pltpu.VMEM((IPW,), jnp.int32),
                   pltpu.VMEM((GC,), jnp.int32),
                   pltpu.VMEM((GC, EMB), jnp.float32)],
    compiler_params=_sc_params())


PB = 2048


def _k4_body(g_ref, wu_ref, bu_ref, ent_ref, out_ref, u_s):
    @pl.when(pl.program_id(0) == 0)
    def _():
        g = g_ref[...]
        t = jnp.sum(jnp.reshape(g, (3, BATCH, HIST, EMB)), axis=2)
        sg = jax.nn.sigmoid(t)
        cc = jnp.concatenate([sg[0], sg[1], sg[2]], axis=-1)
        u = jax.nn.sigmoid(
            jnp.dot(cc, wu_ref[...], preferred_element_type=jnp.float32)
            + bu_ref[...])
        u_s[...] = u

    out_ref[...] = lax.dot_general(
        u_s[...], ent_ref[...], (((1,), (1,)), ((), ())),
        preferred_element_type=jnp.float32)


def _final(g, wu, bu, ent):
    grid = pl.cdiv(N_NODES, PB)
    return pl.pallas_call(
        _k4_body, grid=(grid,),
        in_specs=[pl.BlockSpec((N_IDX, EMB), lambda j: (0, 0)),
                  pl.BlockSpec((3 * EMB, EMB), lambda j: (0, 0)),
                  pl.BlockSpec((1, EMB), lambda j: (0, 0)),
                  pl.BlockSpec((PB, EMB), lambda j: (j, 0))],
        out_specs=pl.BlockSpec((BATCH, PB), lambda j: (0, j)),
        out_shape=_f32(BATCH, N_NODES),
        scratch_shapes=[pltpu.VMEM((BATCH, EMB), jnp.float32)],
    )(g, wu, bu, ent)


def _pad_edges(ei):
    pad = E_PAD - E_PER
    src = jnp.concatenate([ei[0], jnp.zeros((pad,), jnp.int32)])
    dpad = N_NODES + (jnp.arange(pad, dtype=jnp.int32) % 16)
    dst = jnp.concatenate([ei[1], dpad])
    return src, dst


def kernel(edge_index_0, edge_index_1, edge_index_2,
           liked_indices, unknown_indices, disliked_indices,
           embed,
           W1_0, b1_0, W1_1, b1_1, W1_2, b1_2,
           W2_0, b2_0, W2_1, b2_1, W2_2, b2_2,
           Wu, bu):
    edges = [_pad_edges(e) for e in (edge_index_0, edge_index_1, edge_index_2)]
    b1s = [b.reshape(1, HIDDEN) for b in (b1_0, b1_1, b1_2)]
    b2s = [b.reshape(1, EMB) for b in (b2_0, b2_1, b2_2)]
    zrows = jnp.zeros((STG, QW), jnp.float32)
    zcnt = jnp.zeros((TPT,), jnp.float32)

    t1 = _linear1(embed, (W1_0, W1_1, W1_2), b1s)
    agg1 = _make_agg(True)
    s1_0, s1_1, s1_2, cnt = agg1(
        edges[0][0], edges[0][1], edges[1][0], edges[1][1],
        edges[2][0], edges[2][1], *t1, zrows, zcnt)
    cnt = cnt.reshape(3, NPAD)

    t2 = _mid((s1_0, s1_1, s1_2), cnt, (W2_0, W2_1, W2_2), b2s)
    agg2 = _make_agg(False)
    s2_0, s2_1, s2_2 = agg2(
        edges[0][0], edges[0][1], edges[1][0], edges[1][1],
        edges[2][0], edges[2][1], *t2, zrows, zcnt)

    ent = _entities((s2_0, s2_1, s2_2), cnt)

    idxs = jnp.concatenate([liked_indices.reshape(-1),
                            disliked_indices.reshape(-1),
                            unknown_indices.reshape(-1)])
    g = _gather_hist(ent, idxs)
    return _final(g, Wu, bu.reshape(1, EMB), ent)


# speedup vs baseline: 5.5931x; 1.4733x over previous
import functools

import jax
import jax.numpy as jnp
from jax import lax
from jax.experimental import pallas as pl
from jax.experimental.pallas import tpu as pltpu
from jax.experimental.pallas import tpu_sc as plsc

N_NODES = 50000
E_PER = 266667
IN_SIZE, HIDDEN, EMB = 128, 64, 64
BATCH, HIST = 256, 20

NC, NS = 2, 16
CHUNK = 128
N_CH = 131
EW = N_CH * CHUNK
E_PAD = NS * EW
NPAD = 50176
HALF = NPAD // 2
TPT = NPAD // NS
STG = TPT // 4
QW = 16
NQ = 4
FB = 512
NB = HALF // FB

_mesh = functools.partial(
    plsc.VectorSubcoreMesh, core_axis_name="c", subcore_axis_name="s",
    num_cores=NC, num_subcores=NS)

_sc_params = functools.partial(
    pltpu.CompilerParams, use_tc_tiling_on_sc=False)


def _f32(*shape):
    return jax.ShapeDtypeStruct(shape, jnp.float32)


def _k1_body(xl_ref, xr_ref, w0, w1, w2, b0, b1, b2, o0, o1, o2):
    xl = xl_ref[...]
    xr = xr_ref[...]
    for w, b, o in ((w0, b0, o0), (w1, b1, o1), (w2, b2, o2)):
        yl = jnp.dot(xl, w[...], preferred_element_type=jnp.float32) + b[...]
        yr = jnp.dot(xr, w[...], preferred_element_type=jnp.float32) + b[...]
        o[...] = jnp.concatenate([yl, yr], axis=1)


def _linear1(embed, ws, bs):
    fold = pl.BlockSpec((FB, IN_SIZE), lambda i: (i, 0))
    foldr = pl.BlockSpec((FB, IN_SIZE), lambda i: (NB + i, 0))
    return pl.pallas_call(
        _k1_body,
        grid=(NB,),
        in_specs=[fold, foldr]
        + [pl.BlockSpec((IN_SIZE, HIDDEN), lambda i: (0, 0))] * 3
        + [pl.BlockSpec((1, HIDDEN), lambda i: (0, 0))] * 3,
        out_specs=[pl.BlockSpec((FB, 2 * HIDDEN), lambda i: (i, 0))] * 3,
        out_shape=[_f32(HALF, 2 * HIDDEN)] * 3,
    )(embed, embed, *ws, *bs)


def _make_agg(want_cnt):
    outs = [_f32(HALF, 2 * HIDDEN)] * 3
    if want_cnt:
        outs.append(_f32(3 * NPAD))

    scratch = [
        pltpu.VMEM((EW,), jnp.int32),
        pltpu.VMEM((EW,), jnp.int32),
        pltpu.VMEM((CHUNK,), jnp.int32),
        pltpu.VMEM((CHUNK,), jnp.int32),
        pltpu.VMEM((CHUNK, QW), jnp.float32),
        pltpu.VMEM((CHUNK,), jnp.float32),
        pltpu.VMEM((STG, QW), jnp.float32),
        pltpu.VMEM((STG, QW), jnp.float32),
        pltpu.VMEM((TPT,), jnp.float32),
        pltpu.VMEM((TPT,), jnp.float32),
        pltpu.VMEM_SHARED((NPAD, QW), jnp.float32),
        pltpu.VMEM_SHARED((NPAD,), jnp.float32),
    ]

    def body(p0, d0, p1, d1, p2, d2, t0, t1, t2, zrows, zcnt, *rest):
        if want_cnt:
            o0, o1, o2, ocnt = rest[:4]
            rest = rest[4:]
        else:
            o0, o1, o2 = rest[:3]
            rest = rest[3:]
        (preb, dstb, idxv, dstv, rowsv, onesv, stg, wbuf, cbuf, cbuf2,
         acc, cntacc) = rest
        c = lax.axis_index("c")
        s = lax.axis_index("s")
        r0 = s * TPT
        hl = (s >= NS // 2).astype(jnp.int32)
        m0 = r0 - hl * HALF

        if want_cnt:
            for j in range(CHUNK // 16):
                onesv[pl.ds(j * 16, 16)] = jnp.full((16,), 1.0, jnp.float32)

        pres = (p0, p1, p2)
        dsts = (d0, d1, d2)
        tbls = (t0, t1, t2)
        souts = (o0, o1, o2)

        pltpu.sync_copy(zrows, stg)
        if want_cnt:
            pltpu.sync_copy(zcnt, cbuf)

        for e in range(3):
            pltpu.sync_copy(pres[e].at[pl.ds(s * EW, EW)], preb)
            pltpu.sync_copy(dsts[e].at[pl.ds(s * EW, EW)], dstb)

            for p in range(2):
                do_cnt = want_cnt and p == 0
                q = 2 * c + p
                for k in range(TPT // STG):
                    pltpu.sync_copy(stg, acc.at[pl.ds(r0 + k * STG, STG)])
                if do_cnt:
                    @pl.when(c == 0)
                    def _():
                        pltpu.sync_copy(cbuf, cntacc.at[pl.ds(r0, TPT)])
                plsc.subcore_barrier()

                def chunk(k, _, e=e, do_cnt=do_cnt, q=q):
                    kb = k * CHUNK
                    for j in range(CHUNK // 16):
                        idxv[pl.ds(j * 16, 16)] = (
                            preb[pl.ds(kb + j * 16, 16)] + q)
                        dstv[pl.ds(j * 16, 16)] = dstb[pl.ds(kb + j * 16, 16)]
                    pltpu.sync_copy(tbls[e].at[idxv], rowsv)
                    pltpu.sync_copy(rowsv, acc.at[dstv], add=True)
                    if do_cnt:
                        @pl.when(c == 0)
                        def _():
                            pltpu.sync_copy(onesv, cntacc.at[dstv], add=True)
                    return 0

                lax.fori_loop(0, N_CH, chunk, 0)
                plsc.subcore_barrier()

                c0 = QW * q + HIDDEN * hl
                for k in range(TPT // STG):
                    pltpu.sync_copy(acc.at[pl.ds(r0 + k * STG, STG)], wbuf)
                    pltpu.sync_copy(
                        wbuf,
                        souts[e].at[pl.ds(m0 + k * STG, STG), pl.ds(c0, QW)])
                if do_cnt:
                    @pl.when(c == 0)
                    def _(e=e):
                        pltpu.sync_copy(cntacc.at[pl.ds(r0, TPT)], cbuf2)
                        pltpu.sync_copy(cbuf2,
                                        ocnt.at[pl.ds(e * NPAD + r0, TPT)])
                plsc.subcore_barrier()

    return pl.kernel(body, out_type=outs, mesh=_mesh(),
                     scratch_types=scratch,
                     compiler_params=_sc_params())


def _k2_body(s0, s1, s2, cl_ref, cr_ref, w0, w1, w2, b0, b1, b2,
             o0, o1, o2):
    inv_l = 1.0 / jnp.maximum(cl_ref[...], 1.0)
    inv_r = 1.0 / jnp.maximum(cr_ref[...], 1.0)
    h_l = jnp.zeros((FB, HIDDEN), jnp.float32)
    h_r = jnp.zeros((FB, HIDDEN), jnp.float32)
    for e, s_ref in enumerate((s0, s1, s2)):
        blk = s_ref[...]
        h_l = h_l + blk[:, :HIDDEN] * inv_l[e][:, None]
        h_r = h_r + blk[:, HIDDEN:] * inv_r[e][:, None]
    h_l = jnp.where(h_l >= 0, h_l, 0.01 * h_l)
    h_r = jnp.where(h_r >= 0, h_r, 0.01 * h_r)
    for w, b, o in ((w0, b0, o0), (w1, b1, o1), (w2, b2, o2)):
        yl = jnp.dot(h_l, w[...], preferred_element_type=jnp.float32) + b[...]
        yr = jnp.dot(h_r, w[...], preferred_element_type=jnp.float32) + b[...]
        o[...] = jnp.concatenate([yl, yr], axis=1)


def _mid(s1s, cnt, ws, bs):
    fold = pl.BlockSpec((FB, 2 * HIDDEN), lambda i: (i, 0))
    in_specs = [fold] * 3
    in_specs.append(pl.BlockSpec((3, FB), lambda i: (0, i)))
    in_specs.append(pl.BlockSpec((3, FB), lambda i: (0, NB + i)))
    in_specs += [pl.BlockSpec((HIDDEN, HIDDEN), lambda i: (0, 0))] * 3
    in_specs += [pl.BlockSpec((1, HIDDEN), lambda i: (0, 0))] * 3
    return pl.pallas_call(
        _k2_body, grid=(NB,), in_specs=in_specs,
        out_specs=[fold] * 3, out_shape=[_f32(HALF, 2 * HIDDEN)] * 3,
    )(*s1s, cnt, cnt, *ws, *bs)


def _k3_body(s0, s1, s2, cnt_ref, out_ref):
    right = pl.program_id(0) >= NB
    inv = 1.0 / jnp.maximum(cnt_ref[...], 1.0)
    h = jnp.zeros((FB, EMB), jnp.float32)
    for e, s_ref in enumerate((s0, s1, s2)):
        blk = s_ref[...]
        cols = jnp.where(right, blk[:, EMB:], blk[:, :EMB])
        h = h + cols * inv[e][:, None]
    out_ref[...] = jax.nn.sigmoid(h)


def _entities(s2s, cnt):
    fold = pl.BlockSpec(
        (FB, 2 * HIDDEN), lambda i: (jnp.where(i < NB, i, i - NB), 0))
    in_specs = [fold] * 3
    in_specs.append(pl.BlockSpec((3, FB), lambda i: (0, i)))
    return pl.pallas_call(
        _k3_body, grid=(2 * NB,), in_specs=in_specs,
        out_specs=pl.BlockSpec((FB, EMB), lambda i: (i, 0)),
        out_shape=_f32(NPAD, EMB),
    )(*s2s, cnt)


N_IDX = 3 * BATCH * HIST
IPW = N_IDX // (NC * NS)
GC = 96


def _gather_body(ent_h, idx_h, out_h, idxb, idxv, rowsv):
    c = lax.axis_index("c")
    s = lax.axis_index("s")
    base = (s * NC + c) * IPW
    pltpu.sync_copy(idx_h.at[pl.ds(base, IPW)], idxb)
    for k in range(IPW // GC):
        for j in range(GC // 16):
            idxv[pl.ds(j * 16, 16)] = idxb[pl.ds(k * GC + j * 16, 16)]
        pltpu.sync_copy(ent_h.at[idxv], rowsv)
        pltpu.sync_copy(rowsv, out_h.at[pl.ds(base + k * GC, GC)])


_gather_hist = pl.kernel(
    _gather_body, out_type=_f32(N_IDX, EMB), mesh=_mesh(),
    scratch_types=[pltpu.VMEM((IPW,), jnp.int32),
                   pltpu.VMEM((GC,), jnp.int32),
                   pltpu.VMEM((GC, EMB), jnp.float32)],
    compiler_params=_sc_params())


PB = 2048


def _k4_body(g_ref, wu_ref, bu_ref, ent_ref, out_ref, u_s):
    @pl.when(pl.program_id(0) == 0)
    def _():
        g = g_ref[...]
        t = jnp.sum(jnp.reshape(g, (3, BATCH, HIST, EMB)), axis=2)
        sg = jax.nn.sigmoid(t)
        cc = jnp.concatenate([sg[0], sg[1], sg[2]], axis=-1)
        u = jax.nn.sigmoid(
            jnp.dot(cc, wu_ref[...], preferred_element_type=jnp.float32)
            + bu_ref[...])
        u_s[...] = u

    out_ref[...] = lax.dot_general(
        u_s[...], ent_ref[...], (((1,), (1,)), ((), ())),
        preferred_element_type=jnp.float32)


def _final(g, wu, bu, ent):
    grid = pl.cdiv(N_NODES, PB)
    return pl.pallas_call(
        _k4_body, grid=(grid,),
        in_specs=[pl.BlockSpec((N_IDX, EMB), lambda j: (0, 0)),
                  pl.BlockSpec((3 * EMB, EMB), lambda j: (0, 0)),
                  pl.BlockSpec((1, EMB), lambda j: (0, 0)),
                  pl.BlockSpec((PB, EMB), lambda j: (j, 0))],
        out_specs=pl.BlockSpec((BATCH, PB), lambda j: (0, j)),
        out_shape=_f32(BATCH, N_NODES),
        scratch_shapes=[pltpu.VMEM((BATCH, EMB), jnp.float32)],
    )(g, wu, bu, ent)


def _pad_edges(ei):
    pad = E_PAD - E_PER
    src = ei[0]
    pre = src * 8 - jnp.where(src >= HALF, 8 * HALF - 4, 0)
    pre = jnp.concatenate([pre, jnp.zeros((pad,), jnp.int32)])
    dpad = N_NODES + (jnp.arange(pad, dtype=jnp.int32) % 16)
    dst = jnp.concatenate([ei[1], dpad])
    return pre, dst


def kernel(edge_index_0, edge_index_1, edge_index_2,
           liked_indices, unknown_indices, disliked_indices,
           embed,
           W1_0, b1_0, W1_1, b1_1, W1_2, b1_2,
           W2_0, b2_0, W2_1, b2_1, W2_2, b2_2,
           Wu, bu):
    edges = [_pad_edges(e) for e in (edge_index_0, edge_index_1, edge_index_2)]
    b1s = [b.reshape(1, HIDDEN) for b in (b1_0, b1_1, b1_2)]
    b2s = [b.reshape(1, EMB) for b in (b2_0, b2_1, b2_2)]
    zrows = jnp.zeros((STG, QW), jnp.float32)
    zcnt = jnp.zeros((TPT,), jnp.float32)

    def qview(t):
        return t.reshape(8 * HALF, QW)

    t1 = _linear1(embed, (W1_0, W1_1, W1_2), b1s)
    agg1 = _make_agg(True)
    s1_0, s1_1, s1_2, cnt = agg1(
        edges[0][0], edges[0][1], edges[1][0], edges[1][1],
        edges[2][0], edges[2][1], *[qview(t) for t in t1], zrows, zcnt)
    cnt = cnt.reshape(3, NPAD)

    t2 = _mid((s1_0, s1_1, s1_2), cnt, (W2_0, W2_1, W2_2), b2s)
    agg2 = _make_agg(False)
    s2_0, s2_1, s2_2 = agg2(
        edges[0][0], edges[0][1], edges[1][0], edges[1][1],
        edges[2][0], edges[2][1], *[qview(t) for t in t2], zrows, zcnt)

    ent = _entities((s2_0, s2_1, s2_2), cnt)

    idxs = jnp.concatenate([liked_indices.reshape(-1),
                            disliked_indices.reshape(-1),
                            unknown_indices.reshape(-1)])
    g = _gather_hist(ent, idxs)
    return _final(g, Wu, bu.reshape(1, EMB), ent)


# speedup vs baseline: 7.5791x; 1.3551x over previous
import functools

import jax
import jax.numpy as jnp
from jax import lax
from jax.experimental import pallas as pl
from jax.experimental.pallas import tpu as pltpu
from jax.experimental.pallas import tpu_sc as plsc

N_NODES = 50000
E_PER = 266667
IN_SIZE, HIDDEN, EMB = 128, 64, 64
BATCH, HIST = 256, 20

NC, NS = 2, 16
CHUNK = 128
N_CH = 132
EW = N_CH * CHUNK
E_PAD = NS * EW
NPAD = 50176
HALF = NPAD // 2
TPT = NPAD // NS
STG = TPT // 4
QW = 16
NQ = 4
FB = 512
NB = HALF // FB

_mesh = functools.partial(
    plsc.VectorSubcoreMesh, core_axis_name="c", subcore_axis_name="s",
    num_cores=NC, num_subcores=NS)

_sc_params = functools.partial(
    pltpu.CompilerParams, use_tc_tiling_on_sc=False)


def _f32(*shape):
    return jax.ShapeDtypeStruct(shape, jnp.float32)


def _k1_body(xl_ref, xr_ref, w0, w1, w2, b0, b1, b2, o0, o1, o2):
    xl = xl_ref[...]
    xr = xr_ref[...]
    for w, b, o in ((w0, b0, o0), (w1, b1, o1), (w2, b2, o2)):
        yl = jnp.dot(xl, w[...], preferred_element_type=jnp.float32) + b[...]
        yr = jnp.dot(xr, w[...], preferred_element_type=jnp.float32) + b[...]
        o[...] = jnp.concatenate([yl, yr], axis=1)


def _linear1(embed, ws, bs):
    fold = pl.BlockSpec((FB, IN_SIZE), lambda i: (i, 0))
    foldr = pl.BlockSpec((FB, IN_SIZE), lambda i: (NB + i, 0))
    return pl.pallas_call(
        _k1_body,
        grid=(NB,),
        in_specs=[fold, foldr]
        + [pl.BlockSpec((IN_SIZE, HIDDEN), lambda i: (0, 0))] * 3
        + [pl.BlockSpec((1, HIDDEN), lambda i: (0, 0))] * 3,
        out_specs=[pl.BlockSpec((FB, 2 * HIDDEN), lambda i: (i, 0))] * 3,
        out_shape=[_f32(HALF, 2 * HIDDEN)] * 3,
    )(embed, embed, *ws, *bs)


def _make_agg(want_cnt):
    outs = [_f32(HALF, 2 * HIDDEN)] * 3
    if want_cnt:
        outs.append(_f32(3 * NPAD))

    scratch = [
        pltpu.VMEM((N_CH, CHUNK), jnp.int32),
        pltpu.VMEM((N_CH, CHUNK), jnp.int32),
        pltpu.VMEM((CHUNK, QW), jnp.float32),
        pltpu.VMEM((CHUNK, QW), jnp.float32),
        pltpu.VMEM((CHUNK,), jnp.float32),
        pltpu.VMEM((STG, QW), jnp.float32),
        pltpu.VMEM((STG, QW), jnp.float32),
        pltpu.VMEM((TPT,), jnp.float32),
        pltpu.VMEM((TPT,), jnp.float32),
        pltpu.VMEM_SHARED((NPAD, QW), jnp.float32),
        pltpu.VMEM_SHARED((NPAD,), jnp.float32),
        pltpu.SemaphoreType.DMA,
        pltpu.SemaphoreType.DMA,
        pltpu.SemaphoreType.DMA,
        pltpu.SemaphoreType.DMA,
    ]

    def body(*args):
        pre = args[:12]
        d0, d1, d2, t0g, t1g, t2g, zrows, zcnt = args[12:20]
        rest = args[20:]
        if want_cnt:
            o0, o1, o2, ocnt = rest[:4]
            rest = rest[4:]
        else:
            o0, o1, o2 = rest[:3]
            rest = rest[3:]
        (idxb2, dstb2, rowsA, rowsB, onesv, stg, wbuf, cbuf, cbuf2,
         acc, cntacc, gsemA, gsemB, ssemA, ssemB) = rest
        rows = (rowsA, rowsB)
        gsem = (gsemA, gsemB)
        ssem = (ssemA, ssemB)
        c = lax.axis_index("c")
        s = lax.axis_index("s")
        r0 = s * TPT
        hl = (s >= NS // 2).astype(jnp.int32)
        m0 = r0 - hl * HALF

        if want_cnt:
            for j in range(CHUNK // 16):
                onesv[pl.ds(j * 16, 16)] = jnp.full((16,), 1.0, jnp.float32)

        dsts = (d0, d1, d2)
        souts = (o0, o1, o2)

        pltpu.sync_copy(zrows, stg)
        if want_cnt:
            pltpu.sync_copy(zcnt, cbuf)

        for e in range(3):
            pltpu.sync_copy(dsts[e].at[s], dstb2)

            for p in range(2):
                do_cnt = want_cnt and p == 0
                q = 2 * c + p
                @pl.when(c == 0)
                def _(e=e, p=p):
                    pltpu.sync_copy(pre[e * NQ + p].at[s], idxb2)

                @pl.when(c == 1)
                def _(e=e, p=p):
                    pltpu.sync_copy(pre[e * NQ + 2 + p].at[s], idxb2)

                for k in range(TPT // STG):
                    pltpu.sync_copy(stg, acc.at[pl.ds(r0 + k * STG, STG)])
                if do_cnt:
                    @pl.when(c == 0)
                    def _():
                        pltpu.sync_copy(cbuf, cntacc.at[pl.ds(r0, TPT)])
                plsc.subcore_barrier()

                gtbl = (t0g, t1g, t2g)[e]
                pltpu.async_copy(gtbl.at[idxb2.at[0]], rowsA, gsemA)
                pltpu.async_copy(stg.at[pl.ds(0, CHUNK)],
                                 acc.at[dstb2.at[0]], ssemB, add=True)

                def duo(k2, _, e=e, do_cnt=do_cnt, gtbl=gtbl):
                    for u in (0, 1):
                        cc = 2 * k2 + u
                        cur, nxt = u, 1 - u
                        pltpu.make_async_copy(
                            rows[nxt], acc.at[dstb2.at[0]], ssem[nxt]).wait()
                        nc = jnp.minimum(cc + 1, N_CH - 1)
                        pltpu.async_copy(gtbl.at[idxb2.at[nc]], rows[nxt],
                                         gsem[nxt])
                        pltpu.make_async_copy(gtbl.at[idxb2.at[0]],
                                              rows[cur], gsem[cur]).wait()
                        pltpu.async_copy(rows[cur], acc.at[dstb2.at[cc]],
                                         ssem[cur], add=True)
                        if do_cnt:
                            @pl.when(c == 0)
                            def _():
                                pltpu.sync_copy(
                                    onesv, cntacc.at[dstb2.at[cc]], add=True)
                    return 0

                lax.fori_loop(0, N_CH // 2, duo, 0)
                pltpu.make_async_copy(gtbl.at[idxb2.at[0]], rowsA,
                                      gsemA).wait()
                pltpu.make_async_copy(rowsB, acc.at[dstb2.at[0]],
                                      ssemB).wait()
                plsc.subcore_barrier()

                c0 = QW * q + HIDDEN * hl
                for k in range(TPT // STG):
                    pltpu.sync_copy(acc.at[pl.ds(r0 + k * STG, STG)], wbuf)
                    pltpu.sync_copy(
                        wbuf,
                        souts[e].at[pl.ds(m0 + k * STG, STG), pl.ds(c0, QW)])
                if do_cnt:
                    @pl.when(c == 0)
                    def _(e=e):
                        pltpu.sync_copy(cntacc.at[pl.ds(r0, TPT)], cbuf2)
                        pltpu.sync_copy(cbuf2,
                                        ocnt.at[pl.ds(e * NPAD + r0, TPT)])
                plsc.subcore_barrier()

    return pl.kernel(body, out_type=outs, mesh=_mesh(),
                     scratch_types=scratch,
                     compiler_params=_sc_params())


def _k2_body(s0, s1, s2, cl_ref, cr_ref, w0, w1, w2, b0, b1, b2,
             o0, o1, o2):
    inv_l = 1.0 / jnp.maximum(cl_ref[...], 1.0)
    inv_r = 1.0 / jnp.maximum(cr_ref[...], 1.0)
    h_l = jnp.zeros((FB, HIDDEN), jnp.float32)
    h_r = jnp.zeros((FB, HIDDEN), jnp.float32)
    for e, s_ref in enumerate((s0, s1, s2)):
        blk = s_ref[...]
        h_l = h_l + blk[:, :HIDDEN] * inv_l[e][:, None]
        h_r = h_r + blk[:, HIDDEN:] * inv_r[e][:, None]
    h_l = jnp.where(h_l >= 0, h_l, 0.01 * h_l)
    h_r = jnp.where(h_r >= 0, h_r, 0.01 * h_r)
    for w, b, o in ((w0, b0, o0), (w1, b1, o1), (w2, b2, o2)):
        yl = jnp.dot(h_l, w[...], preferred_element_type=jnp.float32) + b[...]
        yr = jnp.dot(h_r, w[...], preferred_element_type=jnp.float32) + b[...]
        o[...] = jnp.concatenate([yl, yr], axis=1)


def _mid(s1s, cnt, ws, bs):
    fold = pl.BlockSpec((FB, 2 * HIDDEN), lambda i: (i, 0))
    in_specs = [fold] * 3
    in_specs.append(pl.BlockSpec((3, FB), lambda i: (0, i)))
    in_specs.append(pl.BlockSpec((3, FB), lambda i: (0, NB + i)))
    in_specs += [pl.BlockSpec((HIDDEN, HIDDEN), lambda i: (0, 0))] * 3
    in_specs += [pl.BlockSpec((1, HIDDEN), lambda i: (0, 0))] * 3
    return pl.pallas_call(
        _k2_body, grid=(NB,), in_specs=in_specs,
        out_specs=[fold] * 3, out_shape=[_f32(HALF, 2 * HIDDEN)] * 3,
    )(*s1s, cnt, cnt, *ws, *bs)


def _k3_body(s0, s1, s2, cnt_ref, out_ref):
    right = pl.program_id(0) >= NB
    inv = 1.0 / jnp.maximum(cnt_ref[...], 1.0)
    h = jnp.zeros((FB, EMB), jnp.float32)
    for e, s_ref in enumerate((s0, s1, s2)):
        blk = s_ref[...]
        cols = jnp.where(right, blk[:, EMB:], blk[:, :EMB])
        h = h + cols * inv[e][:, None]
    out_ref[...] = jax.nn.sigmoid(h)


def _entities(s2s, cnt):
    fold = pl.BlockSpec(
        (FB, 2 * HIDDEN), lambda i: (jnp.where(i < NB, i, i - NB), 0))
    in_specs = [fold] * 3
    in_specs.append(pl.BlockSpec((3, FB), lambda i: (0, i)))
    return pl.pallas_call(
        _k3_body, grid=(2 * NB,), in_specs=in_specs,
        out_specs=pl.BlockSpec((FB, EMB), lambda i: (i, 0)),
        out_shape=_f32(NPAD, EMB),
    )(*s2s, cnt)


N_IDX = 3 * BATCH * HIST
IPW = N_IDX // (NC * NS)
GC = 96


def _gather_body(ent_h, idx_h, out_h, idxb, idxv, rowsv):
    c = lax.axis_index("c")
    s = lax.axis_index("s")
    base = (s * NC + c) * IPW
    pltpu.sync_copy(idx_h.at[pl.ds(base, IPW)], idxb)
    for k in range(IPW // GC):
        for j in range(GC // 16):
            idxv[pl.ds(j * 16, 16)] = idxb[pl.ds(k * GC + j * 16, 16)]
        pltpu.sync_copy(ent_h.at[idxv], rowsv)
        pltpu.sync_copy(rowsv, out_h.at[pl.ds(base + k * GC, GC)])


_gather_hist = pl.kernel(
    _gather_body, out_type=_f32(N_IDX, EMB), mesh=_mesh(),
    scratch_types=[pltpu.VMEM((IPW,), jnp.int32),
                   pltpu.VMEM((GC,), jnp.int32),
                   pltpu.VMEM((GC, EMB), jnp.float32)],
    compiler_params=_sc_params())


PB = 2048


def _k4_body(g_ref, wu_ref, bu_ref, ent_ref, out_ref, u_s):
    @pl.when(pl.program_id(0) == 0)
    def _():
        g = g_ref[...]
        t = jnp.sum(jnp.reshape(g, (3, BATCH, HIST, EMB)), axis=2)
        sg = jax.nn.sigmoid(t)
        cc = jnp.concatenate([sg[0], sg[1], sg[2]], axis=-1)
        u = jax.nn.sigmoid(
            jnp.dot(cc, wu_ref[...], preferred_element_type=jnp.float32)
            + bu_ref[...])
        u_s[...] = u

    out_ref[...] = lax.dot_general(
        u_s[...], ent_ref[...], (((1,), (1,)), ((), ())),
        preferred_element_type=jnp.float32)


def _final(g, wu, bu, ent):
    grid = pl.cdiv(N_NODES, PB)
    return pl.pallas_call(
        _k4_body, grid=(grid,),
        in_specs=[pl.BlockSpec((N_IDX, EMB), lambda j: (0, 0)),
                  pl.BlockSpec((3 * EMB, EMB), lambda j: (0, 0)),
                  pl.BlockSpec((1, EMB), lambda j: (0, 0)),
                  pl.BlockSpec((PB, EMB), lambda j: (j, 0))],
        out_specs=pl.BlockSpec((BATCH, PB), lambda j: (0, j)),
        out_shape=_f32(BATCH, N_NODES),
        scratch_shapes=[pltpu.VMEM((BATCH, EMB), jnp.float32)],
    )(g, wu, bu, ent)


def _pad_edges(ei):
    pad = E_PAD - E_PER
    src = ei[0]
    pre = src * 8 - jnp.where(src >= HALF, 8 * HALF - 4, 0)
    pre = jnp.concatenate([pre, jnp.zeros((pad,), jnp.int32)])
    pres = [(pre + q).reshape(NS, N_CH, CHUNK) for q in range(NQ)]
    dpad = N_NODES + (jnp.arange(pad, dtype=jnp.int32) % 16)
    dst = jnp.concatenate([ei[1], dpad]).reshape(NS, N_CH, CHUNK)
    return pres, dst


def kernel(edge_index_0, edge_index_1, edge_index_2,
           liked_indices, unknown_indices, disliked_indices,
           embed,
           W1_0, b1_0, W1_1, b1_1, W1_2, b1_2,
           W2_0, b2_0, W2_1, b2_1, W2_2, b2_2,
           Wu, bu):
    edges = [_pad_edges(e) for e in (edge_index_0, edge_index_1, edge_index_2)]
    b1s = [b.reshape(1, HIDDEN) for b in (b1_0, b1_1, b1_2)]
    b2s = [b.reshape(1, EMB) for b in (b2_0, b2_1, b2_2)]
    zrows = jnp.zeros((STG, QW), jnp.float32)
    zcnt = jnp.zeros((TPT,), jnp.float32)

    def qview(t):
        return t.reshape(8 * HALF, QW)

    pres = [p for e in range(3) for p in edges[e][0]]
    dsts = [edges[e][1] for e in range(3)]

    t1 = _linear1(embed, (W1_0, W1_1, W1_2), b1s)
    agg1 = _make_agg(True)
    s1_0, s1_1, s1_2, cnt = agg1(
        *pres, *dsts, *[qview(t) for t in t1], zrows, zcnt)
    cnt = cnt.reshape(3, NPAD)

    t2 = _mid((s1_0, s1_1, s1_2), cnt, (W2_0, W2_1, W2_2), b2s)
    agg2 = _make_agg(False)
    s2_0, s2_1, s2_2 = agg2(
        *pres, *dsts, *[qview(t) for t in t2], zrows, zcnt)

    ent = _entities((s2_0, s2_1, s2_2), cnt)

    idxs = jnp.concatenate([liked_indices.reshape(-1),
                            disliked_indices.reshape(-1),
                            unknown_indices.reshape(-1)])
    g = _gather_hist(ent, idxs)
    return _final(g, Wu, bu.reshape(1, EMB), ent)


# speedup vs baseline: 8.0379x; 1.0605x over previous
import functools

import jax
import jax.numpy as jnp
from jax import lax
from jax.experimental import pallas as pl
from jax.experimental.pallas import tpu as pltpu
from jax.experimental.pallas import tpu_sc as plsc

N_NODES = 50000
E_PER = 266667
IN_SIZE, HIDDEN, EMB = 128, 64, 64
BATCH, HIST = 256, 20

NC, NS = 2, 16
CHUNK = 128
N_CH = 132
EW = N_CH * CHUNK
E_PAD = NS * EW
NPAD = 50176
HALF = NPAD // 2
TPT = NPAD // NS
STG = TPT // 4
QW = 16
NQ = 4
FB = 512
NB = HALF // FB

_mesh = functools.partial(
    plsc.VectorSubcoreMesh, core_axis_name="c", subcore_axis_name="s",
    num_cores=NC, num_subcores=NS)

_sc_params = functools.partial(
    pltpu.CompilerParams, use_tc_tiling_on_sc=False)


def _f32(*shape):
    return jax.ShapeDtypeStruct(shape, jnp.float32)


def _k1_body(xl_ref, xr_ref, w0, w1, w2, b0, b1, b2, o0, o1, o2):
    xl = xl_ref[...]
    xr = xr_ref[...]
    for w, b, o in ((w0, b0, o0), (w1, b1, o1), (w2, b2, o2)):
        yl = jnp.dot(xl, w[...], preferred_element_type=jnp.float32) + b[...]
        yr = jnp.dot(xr, w[...], preferred_element_type=jnp.float32) + b[...]
        o[...] = jnp.concatenate([yl, yr], axis=1)


def _linear1(embed, ws, bs):
    fold = pl.BlockSpec((FB, IN_SIZE), lambda i: (i, 0))
    foldr = pl.BlockSpec((FB, IN_SIZE), lambda i: (NB + i, 0))
    return pl.pallas_call(
        _k1_body,
        grid=(NB,),
        in_specs=[fold, foldr]
        + [pl.BlockSpec((IN_SIZE, HIDDEN), lambda i: (0, 0))] * 3
        + [pl.BlockSpec((1, HIDDEN), lambda i: (0, 0))] * 3,
        out_specs=[pl.BlockSpec((FB, 2 * HIDDEN), lambda i: (i, 0))] * 3,
        out_shape=[_f32(HALF, 2 * HIDDEN)] * 3,
    )(embed, embed, *ws, *bs)


def _make_agg(want_cnt):
    outs = [_f32(HALF, 2 * HIDDEN)] * 3
    if want_cnt:
        outs.append(_f32(3 * NPAD))

    scratch = [
        pltpu.VMEM((N_CH, CHUNK), jnp.int32),
        pltpu.VMEM((N_CH, CHUNK), jnp.int32),
        pltpu.VMEM((CHUNK, QW), jnp.float32),
        pltpu.VMEM((CHUNK, QW), jnp.float32),
        pltpu.VMEM((CHUNK, QW), jnp.float32),
        pltpu.VMEM((CHUNK, QW), jnp.float32),
        pltpu.VMEM((CHUNK,), jnp.float32),
        pltpu.VMEM((STG, QW), jnp.float32),
        pltpu.VMEM((STG, QW), jnp.float32),
        pltpu.VMEM((TPT,), jnp.float32),
        pltpu.VMEM((TPT,), jnp.float32),
        pltpu.VMEM_SHARED((NPAD, QW), jnp.float32),
        pltpu.VMEM_SHARED((NPAD,), jnp.float32),
    ] + [pltpu.SemaphoreType.DMA] * 8

    def body(*args):
        pre = args[:12]
        d0, d1, d2, t0g, t1g, t2g, zrows, zcnt = args[12:20]
        rest = args[20:]
        if want_cnt:
            o0, o1, o2, ocnt = rest[:4]
            rest = rest[4:]
        else:
            o0, o1, o2 = rest[:3]
            rest = rest[3:]
        (idxb2, dstb2, rows0, rows1, rows2, rows3, onesv, stg, wbuf,
         cbuf, cbuf2, acc, cntacc, *sems) = rest
        rows = (rows0, rows1, rows2, rows3)
        gsem = tuple(sems[:4])
        ssem = tuple(sems[4:])
        c = lax.axis_index("c")
        s = lax.axis_index("s")
        r0 = s * TPT
        hl = (s >= NS // 2).astype(jnp.int32)
        m0 = r0 - hl * HALF

        if want_cnt:
            for j in range(CHUNK // 16):
                onesv[pl.ds(j * 16, 16)] = jnp.full((16,), 1.0, jnp.float32)

        dsts = (d0, d1, d2)
        souts = (o0, o1, o2)

        pltpu.sync_copy(zrows, stg)
        if want_cnt:
            pltpu.sync_copy(zcnt, cbuf)

        for e in range(3):
            pltpu.sync_copy(dsts[e].at[s], dstb2)

            for p in range(2):
                do_cnt = want_cnt and p == 0
                q = 2 * c + p
                @pl.when(c == 0)
                def _(e=e, p=p):
                    pltpu.sync_copy(pre[e * NQ + p].at[s], idxb2)

                @pl.when(c == 1)
                def _(e=e, p=p):
                    pltpu.sync_copy(pre[e * NQ + 2 + p].at[s], idxb2)

                for k in range(TPT // STG):
                    pltpu.sync_copy(stg, acc.at[pl.ds(r0 + k * STG, STG)])
                if do_cnt:
                    @pl.when(c == 0)
                    def _():
                        pltpu.sync_copy(cbuf, cntacc.at[pl.ds(r0, TPT)])
                plsc.subcore_barrier()

                gtbl = (t0g, t1g, t2g)[e]
                pltpu.async_copy(gtbl.at[idxb2.at[0]], rows[0], gsem[0])
                for v in (1, 2, 3):
                    pltpu.async_copy(stg.at[pl.ds(0, CHUNK)],
                                     acc.at[dstb2.at[0]], ssem[v], add=True)

                def quad(k4, _, e=e, do_cnt=do_cnt, gtbl=gtbl):
                    for u in range(4):
                        cc = 4 * k4 + u
                        cur, nxt = u, (u + 1) % 4
                        pltpu.make_async_copy(
                            rows[nxt], acc.at[dstb2.at[0]], ssem[nxt]).wait()
                        nc = jnp.minimum(cc + 1, N_CH - 1)
                        pltpu.async_copy(gtbl.at[idxb2.at[nc]], rows[nxt],
                                         gsem[nxt])
                        pltpu.make_async_copy(gtbl.at[idxb2.at[0]],
                                              rows[cur], gsem[cur]).wait()
                        pltpu.async_copy(rows[cur], acc.at[dstb2.at[cc]],
                                         ssem[cur], add=True)
                        if do_cnt:
                            @pl.when(c == 0)
                            def _():
                                pltpu.sync_copy(
                                    onesv, cntacc.at[dstb2.at[cc]], add=True)
                    return 0

                lax.fori_loop(0, N_CH // 4, quad, 0)
                pltpu.make_async_copy(gtbl.at[idxb2.at[0]], rows[0],
                                      gsem[0]).wait()
                for v in (1, 2, 3):
                    pltpu.make_async_copy(rows[v], acc.at[dstb2.at[0]],
                                          ssem[v]).wait()
                plsc.subcore_barrier()

                c0 = QW * q + HIDDEN * hl
                for k in range(TPT // STG):
                    pltpu.sync_copy(acc.at[pl.ds(r0 + k * STG, STG)], wbuf)
                    pltpu.sync_copy(
                        wbuf,
                        souts[e].at[pl.ds(m0 + k * STG, STG), pl.ds(c0, QW)])
                if do_cnt:
                    @pl.when(c == 0)
                    def _(e=e):
                        pltpu.sync_copy(cntacc.at[pl.ds(r0, TPT)], cbuf2)
                        pltpu.sync_copy(cbuf2,
                                        ocnt.at[pl.ds(e * NPAD + r0, TPT)])
                plsc.subcore_barrier()

    return pl.kernel(body, out_type=outs, mesh=_mesh(),
                     scratch_types=scratch,
                     compiler_params=_sc_params())


def _k2_body(s0, s1, s2, cl_ref, cr_ref, w0, w1, w2, b0, b1, b2,
             o0, o1, o2):
    inv_l = 1.0 / jnp.maximum(cl_ref[...], 1.0)
    inv_r = 1.0 / jnp.maximum(cr_ref[...], 1.0)
    h_l = jnp.zeros((FB, HIDDEN), jnp.float32)
    h_r = jnp.zeros((FB, HIDDEN), jnp.float32)
    for e, s_ref in enumerate((s0, s1, s2)):
        blk = s_ref[...]
        h_l = h_l + blk[:, :HIDDEN] * inv_l[e][:, None]
        h_r = h_r + blk[:, HIDDEN:] * inv_r[e][:, None]
    h_l = jnp.where(h_l >= 0, h_l, 0.01 * h_l)
    h_r = jnp.where(h_r >= 0, h_r, 0.01 * h_r)
    for w, b, o in ((w0, b0, o0), (w1, b1, o1), (w2, b2, o2)):
        yl = jnp.dot(h_l, w[...], preferred_element_type=jnp.float32) + b[...]
        yr = jnp.dot(h_r, w[...], preferred_element_type=jnp.float32) + b[...]
        o[...] = jnp.concatenate([yl, yr], axis=1)


def _mid(s1s, cnt, ws, bs):
    fold = pl.BlockSpec((FB, 2 * HIDDEN), lambda i: (i, 0))
    in_specs = [fold] * 3
    in_specs.append(pl.BlockSpec((3, FB), lambda i: (0, i)))
    in_specs.append(pl.BlockSpec((3, FB), lambda i: (0, NB + i)))
    in_specs += [pl.BlockSpec((HIDDEN, HIDDEN), lambda i: (0, 0))] * 3
    in_specs += [pl.BlockSpec((1, HIDDEN), lambda i: (0, 0))] * 3
    return pl.pallas_call(
        _k2_body, grid=(NB,), in_specs=in_specs,
        out_specs=[fold] * 3, out_shape=[_f32(HALF, 2 * HIDDEN)] * 3,
    )(*s1s, cnt, cnt, *ws, *bs)


def _k3_body(s0, s1, s2, cnt_ref, out_ref):
    right = pl.program_id(0) >= NB
    inv = 1.0 / jnp.maximum(cnt_ref[...], 1.0)
    h = jnp.zeros((FB, EMB), jnp.float32)
    for e, s_ref in enumerate((s0, s1, s2)):
        blk = s_ref[...]
        cols = jnp.where(right, blk[:, EMB:], blk[:, :EMB])
        h = h + cols * inv[e][:, None]
    out_ref[...] = jax.nn.sigmoid(h)


def _entities(s2s, cnt):
    fold = pl.BlockSpec(
        (FB, 2 * HIDDEN), lambda i: (jnp.where(i < NB, i, i - NB), 0))
    in_specs = [fold] * 3
    in_specs.append(pl.BlockSpec((3, FB), lambda i: (0, i)))
    return pl.pallas_call(
        _k3_body, grid=(2 * NB,), in_specs=in_specs,
        out_specs=pl.BlockSpec((FB, EMB), lambda i: (i, 0)),
        out_shape=_f32(NPAD, EMB),
    )(*s2s, cnt)


N_IDX = 3 * BATCH * HIST
IPW = N_IDX // (NC * NS)
GC = 96


def _gather_body(ent_h, idx_h, out_h, idxb, idxv, rowsv):
    c = lax.axis_index("c")
    s = lax.axis_index("s")
    base = (s * NC + c) * IPW
    pltpu.sync_copy(idx_h.at[pl.ds(base, IPW)], idxb)
    for k in range(IPW // GC):
        for j in range(GC // 16):
            idxv[pl.ds(j * 16, 16)] = idxb[pl.ds(k * GC + j * 16, 16)]
        pltpu.sync_copy(ent_h.at[idxv], rowsv)
        pltpu.sync_copy(rowsv, out_h.at[pl.ds(base + k * GC, GC)])


_gather_hist = pl.kernel(
    _gather_body, out_type=_f32(N_IDX, EMB), mesh=_mesh(),
    scratch_types=[pltpu.VMEM((IPW,), jnp.int32),
                   pltpu.VMEM((GC,), jnp.int32),
                   pltpu.VMEM((GC, EMB), jnp.float32)],
    compiler_params=_sc_params())


PB = 2048


def _k4_body(g_ref, wu_ref, bu_ref, ent_ref, out_ref, u_s):
    @pl.when(pl.program_id(0) == 0)
    def _():
        g = g_ref[...]
        t = jnp.sum(jnp.reshape(g, (3, BATCH, HIST, EMB)), axis=2)
        sg = jax.nn.sigmoid(t)
        cc = jnp.concatenate([sg[0], sg[1], sg[2]], axis=-1)
        u = jax.nn.sigmoid(
            jnp.dot(cc, wu_ref[...], preferred_element_type=jnp.float32)
            + bu_ref[...])
        u_s[...] = u

    out_ref[...] = lax.dot_general(
        u_s[...], ent_ref[...], (((1,), (1,)), ((), ())),
        preferred_element_type=jnp.float32)


def _final(g, wu, bu, ent):
    grid = pl.cdiv(N_NODES, PB)
    return pl.pallas_call(
        _k4_body, grid=(grid,),
        in_specs=[pl.BlockSpec((N_IDX, EMB), lambda j: (0, 0)),
                  pl.BlockSpec((3 * EMB, EMB), lambda j: (0, 0)),
                  pl.BlockSpec((1, EMB), lambda j: (0, 0)),
                  pl.BlockSpec((PB, EMB), lambda j: (j, 0))],
        out_specs=pl.BlockSpec((BATCH, PB), lambda j: (0, j)),
        out_shape=_f32(BATCH, N_NODES),
        scratch_shapes=[pltpu.VMEM((BATCH, EMB), jnp.float32)],
    )(g, wu, bu, ent)


def _pad_edges(ei):
    pad = E_PAD - E_PER
    src = ei[0]
    pre = src * 8 - jnp.where(src >= HALF, 8 * HALF - 4, 0)
    pre = jnp.concatenate([pre, jnp.zeros((pad,), jnp.int32)])
    pres = [(pre + q).reshape(NS, N_CH, CHUNK) for q in range(NQ)]
    dpad = N_NODES + (jnp.arange(pad, dtype=jnp.int32) % 16)
    dst = jnp.concatenate([ei[1], dpad]).reshape(NS, N_CH, CHUNK)
    return pres, dst


def kernel(edge_index_0, edge_index_1, edge_index_2,
           liked_indices, unknown_indices, disliked_indices,
           embed,
           W1_0, b1_0, W1_1, b1_1, W1_2, b1_2,
           W2_0, b2_0, W2_1, b2_1, W2_2, b2_2,
           Wu, bu):
    edges = [_pad_edges(e) for e in (edge_index_0, edge_index_1, edge_index_2)]
    b1s = [b.reshape(1, HIDDEN) for b in (b1_0, b1_1, b1_2)]
    b2s = [b.reshape(1, EMB) for b in (b2_0, b2_1, b2_2)]
    zrows = jnp.zeros((STG, QW), jnp.float32)
    zcnt = jnp.zeros((TPT,), jnp.float32)

    def qview(t):
        return t.reshape(8 * HALF, QW)

    pres = [p for e in range(3) for p in edges[e][0]]
    dsts = [edges[e][1] for e in range(3)]

    t1 = _linear1(embed, (W1_0, W1_1, W1_2), b1s)
    agg1 = _make_agg(True)
    s1_0, s1_1, s1_2, cnt = agg1(
        *pres, *dsts, *[qview(t) for t in t1], zrows, zcnt)
    cnt = cnt.reshape(3, NPAD)

    t2 = _mid((s1_0, s1_1, s1_2), cnt, (W2_0, W2_1, W2_2), b2s)
    agg2 = _make_agg(False)
    s2_0, s2_1, s2_2 = agg2(
        *pres, *dsts, *[qview(t) for t in t2], zrows, zcnt)

    ent = _entities((s2_0, s2_1, s2_2), cnt)

    idxs = jnp.concatenate([liked_indices.reshape(-1),
                            disliked_indices.reshape(-1),
                            unknown_indices.reshape(-1)])
    g = _gather_hist(ent, idxs)
    return _final(g, Wu, bu.reshape(1, EMB), ent)


# speedup vs baseline: 8.4302x; 1.0488x over previous
import functools

import jax
import jax.numpy as jnp
from jax import lax
from jax.experimental import pallas as pl
from jax.experimental.pallas import tpu as pltpu
from jax.experimental.pallas import tpu_sc as plsc

N_NODES = 50000
E_PER = 266667
IN_SIZE, HIDDEN, EMB = 128, 64, 64
BATCH, HIST = 256, 20

NC, NS = 2, 16
CHUNK = 128
N_CH = 132
EW = N_CH * CHUNK
E_PAD = NS * EW
NPAD = 50176
HALF = NPAD // 2
TPT = NPAD // NS
STG = TPT // 4
QW = 16
NQ = 4
FB = 512
NB = HALF // FB

_mesh = functools.partial(
    plsc.VectorSubcoreMesh, core_axis_name="c", subcore_axis_name="s",
    num_cores=NC, num_subcores=NS)

_sc_params = functools.partial(
    pltpu.CompilerParams, use_tc_tiling_on_sc=False)


def _f32(*shape):
    return jax.ShapeDtypeStruct(shape, jnp.float32)


def _k1_body(xl_ref, xr_ref, w0, w1, w2, b0, b1, b2, o0, o1, o2):
    xl = xl_ref[...]
    xr = xr_ref[...]
    for w, b, o in ((w0, b0, o0), (w1, b1, o1), (w2, b2, o2)):
        yl = jnp.dot(xl, w[...], preferred_element_type=jnp.float32) + b[...]
        yr = jnp.dot(xr, w[...], preferred_element_type=jnp.float32) + b[...]
        o[...] = jnp.concatenate([yl, yr], axis=1)


def _linear1(embed, ws, bs):
    fold = pl.BlockSpec((FB, IN_SIZE), lambda i: (i, 0))
    foldr = pl.BlockSpec((FB, IN_SIZE), lambda i: (NB + i, 0))
    return pl.pallas_call(
        _k1_body,
        grid=(NB,),
        in_specs=[fold, foldr]
        + [pl.BlockSpec((IN_SIZE, HIDDEN), lambda i: (0, 0))] * 3
        + [pl.BlockSpec((1, HIDDEN), lambda i: (0, 0))] * 3,
        out_specs=[pl.BlockSpec((FB, 2 * HIDDEN), lambda i: (i, 0))] * 3,
        out_shape=[_f32(HALF, 2 * HIDDEN)] * 3,
    )(embed, embed, *ws, *bs)


def _make_agg(want_cnt):
    outs = [_f32(HALF, 2 * HIDDEN)] * 3
    if want_cnt:
        outs.append(_f32(3 * NPAD))

    scratch = [
        pltpu.VMEM((N_CH, CHUNK), jnp.int32),
        pltpu.VMEM((N_CH, CHUNK), jnp.int32),
        pltpu.VMEM((CHUNK, QW), jnp.float32),
        pltpu.VMEM((CHUNK, QW), jnp.float32),
        pltpu.VMEM((CHUNK, QW), jnp.float32),
        pltpu.VMEM((CHUNK, QW), jnp.float32),
        pltpu.VMEM((CHUNK,), jnp.float32),
        pltpu.VMEM((STG, QW), jnp.float32),
        pltpu.VMEM((STG, QW), jnp.float32),
        pltpu.VMEM((TPT,), jnp.float32),
        pltpu.VMEM((TPT,), jnp.float32),
        pltpu.VMEM_SHARED((NPAD, QW), jnp.float32),
        pltpu.VMEM_SHARED((NPAD,), jnp.float32),
    ] + [pltpu.SemaphoreType.DMA] * 8

    def body(*args):
        pre = args[:12]
        d0, d1, d2, t0g, t1g, t2g, zrows, zcnt = args[12:20]
        rest = args[20:]
        if want_cnt:
            o0, o1, o2, ocnt = rest[:4]
            rest = rest[4:]
        else:
            o0, o1, o2 = rest[:3]
            rest = rest[3:]
        (idxb2, dstb2, rows0, rows1, rows2, rows3, onesv, stg, wbuf,
         cbuf, cbuf2, acc, cntacc, *sems) = rest
        rows = (rows0, rows1, rows2, rows3)
        gsem = tuple(sems[:4])
        ssem = tuple(sems[4:])
        c = lax.axis_index("c")
        s = lax.axis_index("s")
        r0 = s * TPT
        hl = (s >= NS // 2).astype(jnp.int32)
        m0 = r0 - hl * HALF

        if want_cnt:
            for j in range(CHUNK // 16):
                onesv[pl.ds(j * 16, 16)] = jnp.full((16,), 1.0, jnp.float32)

        dsts = (d0, d1, d2)
        souts = (o0, o1, o2)

        pltpu.sync_copy(zrows, stg)
        if want_cnt:
            pltpu.sync_copy(zcnt, cbuf)

        for e in range(3):
            pltpu.sync_copy(dsts[e].at[s], dstb2)

            for p in range(2):
                do_cnt = want_cnt and p == 0
                q = 2 * c + p
                @pl.when(c == 0)
                def _(e=e, p=p):
                    pltpu.sync_copy(pre[e * NQ + p].at[s], idxb2)

                @pl.when(c == 1)
                def _(e=e, p=p):
                    pltpu.sync_copy(pre[e * NQ + 2 + p].at[s], idxb2)

                for k in range(TPT // STG):
                    pltpu.sync_copy(stg, acc.at[pl.ds(r0 + k * STG, STG)])
                if do_cnt:
                    @pl.when(c == 0)
                    def _():
                        pltpu.sync_copy(cbuf, cntacc.at[pl.ds(r0, TPT)])
                plsc.subcore_barrier()

                gtbl = (t0g, t1g, t2g)[e]
                pltpu.async_copy(gtbl.at[idxb2.at[0]], rows[0], gsem[0])
                for v in (1, 2, 3):
                    pltpu.async_copy(stg.at[pl.ds(0, CHUNK)],
                                     acc.at[dstb2.at[0]], ssem[v], add=True)

                def quad(k4, _, e=e, do_cnt=do_cnt, gtbl=gtbl):
                    for u in range(4):
                        cc = 4 * k4 + u
                        cur, nxt = u, (u + 1) % 4
                        pltpu.make_async_copy(
                            rows[nxt], acc.at[dstb2.at[0]], ssem[nxt]).wait()
                        nc = jnp.minimum(cc + 1, N_CH - 1)
                        pltpu.async_copy(gtbl.at[idxb2.at[nc]], rows[nxt],
                                         gsem[nxt])
                        pltpu.make_async_copy(gtbl.at[idxb2.at[0]],
                                              rows[cur], gsem[cur]).wait()
                        pltpu.async_copy(rows[cur], acc.at[dstb2.at[cc]],
                                         ssem[cur], add=True)
                        if do_cnt:
                            @pl.when(c == 0)
                            def _():
                                pltpu.sync_copy(
                                    onesv, cntacc.at[dstb2.at[cc]], add=True)
                    return 0

                lax.fori_loop(0, N_CH // 4, quad, 0)
                pltpu.make_async_copy(gtbl.at[idxb2.at[0]], rows[0],
                                      gsem[0]).wait()
                for v in (1, 2, 3):
                    pltpu.make_async_copy(rows[v], acc.at[dstb2.at[0]],
                                          ssem[v]).wait()
                plsc.subcore_barrier()

                c0 = QW * q + HIDDEN * hl
                for k in range(TPT // STG):
                    pltpu.sync_copy(acc.at[pl.ds(r0 + k * STG, STG)], wbuf)
                    pltpu.sync_copy(
                        wbuf,
                        souts[e].at[pl.ds(m0 + k * STG, STG), pl.ds(c0, QW)])
                if do_cnt:
                    @pl.when(c == 0)
                    def _(e=e):
                        pltpu.sync_copy(cntacc.at[pl.ds(r0, TPT)], cbuf2)
                        pltpu.sync_copy(cbuf2,
                                        ocnt.at[pl.ds(e * NPAD + r0, TPT)])
                plsc.subcore_barrier()

    return pl.kernel(body, out_type=outs, mesh=_mesh(),
                     scratch_types=scratch,
                     compiler_params=_sc_params())


def _k2_body(s0, s1, s2, cl_ref, cr_ref, w0, w1, w2, b0, b1, b2,
             o0, o1, o2):
    inv_l = 1.0 / jnp.maximum(cl_ref[...], 1.0)
    inv_r = 1.0 / jnp.maximum(cr_ref[...], 1.0)
    h_l = jnp.zeros((FB, HIDDEN), jnp.float32)
    h_r = jnp.zeros((FB, HIDDEN), jnp.float32)
    for e, s_ref in enumerate((s0, s1, s2)):
        blk = s_ref[...]
        h_l = h_l + blk[:, :HIDDEN] * inv_l[e][:, None]
        h_r = h_r + blk[:, HIDDEN:] * inv_r[e][:, None]
    h_l = jnp.where(h_l >= 0, h_l, 0.01 * h_l)
    h_r = jnp.where(h_r >= 0, h_r, 0.01 * h_r)
    for w, b, o in ((w0, b0, o0), (w1, b1, o1), (w2, b2, o2)):
        yl = jnp.dot(h_l, w[...], preferred_element_type=jnp.float32) + b[...]
        yr = jnp.dot(h_r, w[...], preferred_element_type=jnp.float32) + b[...]
        o[...] = jnp.concatenate([yl, yr], axis=1)


def _mid(s1s, cnt, ws, bs):
    fold = pl.BlockSpec((FB, 2 * HIDDEN), lambda i: (i, 0))
    in_specs = [fold] * 3
    in_specs.append(pl.BlockSpec((3, FB), lambda i: (0, i)))
    in_specs.append(pl.BlockSpec((3, FB), lambda i: (0, NB + i)))
    in_specs += [pl.BlockSpec((HIDDEN, HIDDEN), lambda i: (0, 0))] * 3
    in_specs += [pl.BlockSpec((1, HIDDEN), lambda i: (0, 0))] * 3
    return pl.pallas_call(
        _k2_body, grid=(NB,), in_specs=in_specs,
        out_specs=[fold] * 3, out_shape=[_f32(HALF, 2 * HIDDEN)] * 3,
    )(*s1s, cnt, cnt, *ws, *bs)


def _k3_body(s0, s1, s2, cl_ref, cr_ref, out_ref):
    inv_l = 1.0 / jnp.maximum(cl_ref[...], 1.0)
    inv_r = 1.0 / jnp.maximum(cr_ref[...], 1.0)
    h_l = jnp.zeros((FB, EMB), jnp.float32)
    h_r = jnp.zeros((FB, EMB), jnp.float32)
    for e, s_ref in enumerate((s0, s1, s2)):
        blk = s_ref[...]
        h_l = h_l + blk[:, :EMB] * inv_l[e][:, None]
        h_r = h_r + blk[:, EMB:] * inv_r[e][:, None]
    out_ref[...] = jnp.concatenate(
        [jax.nn.sigmoid(h_l), jax.nn.sigmoid(h_r)], axis=1)


def _entities(s2s, cnt):
    fold = pl.BlockSpec((FB, 2 * HIDDEN), lambda i: (i, 0))
    in_specs = [fold] * 3
    in_specs.append(pl.BlockSpec((3, FB), lambda i: (0, i)))
    in_specs.append(pl.BlockSpec((3, FB), lambda i: (0, NB + i)))
    return pl.pallas_call(
        _k3_body, grid=(NB,), in_specs=in_specs,
        out_specs=fold, out_shape=_f32(HALF, 2 * HIDDEN),
    )(*s2s, cnt, cnt)


N_IDX = 3 * BATCH * HIST
IPW = N_IDX // (NC * NS)
GC = 96


def _gather_body(ent_h, idx_h, out_h, idxb, idxv, rowsv):
    c = lax.axis_index("c")
    s = lax.axis_index("s")
    base = (s * NC + c) * IPW
    pltpu.sync_copy(idx_h.at[pl.ds(base, IPW)], idxb)
    for k in range(IPW // GC):
        for j in range(GC // 16):
            idxv[pl.ds(j * 16, 16)] = idxb[pl.ds(k * GC + j * 16, 16)]
        pltpu.sync_copy(ent_h.at[idxv], rowsv)
        pltpu.sync_copy(rowsv, out_h.at[pl.ds(base + k * GC, GC)])


_gather_hist = pl.kernel(
    _gather_body, out_type=_f32(N_IDX, EMB), mesh=_mesh(),
    scratch_types=[pltpu.VMEM((IPW,), jnp.int32),
                   pltpu.VMEM((GC,), jnp.int32),
                   pltpu.VMEM((GC, EMB), jnp.float32)],
    compiler_params=_sc_params())


PB = 2048


def _k4_body(g_ref, wu_ref, bu_ref, ent_ref, out_ref, u_s):
    @pl.when((pl.program_id(0) == 0) & (pl.program_id(1) == 0))
    def _():
        g = g_ref[...]
        t = jnp.sum(jnp.reshape(g, (3, BATCH, HIST, EMB)), axis=2)
        sg = jax.nn.sigmoid(t)
        cc = jnp.concatenate([sg[0], sg[1], sg[2]], axis=-1)
        u = jax.nn.sigmoid(
            jnp.dot(cc, wu_ref[...], preferred_element_type=jnp.float32)
            + bu_ref[...])
        u_s[...] = u

    blk = ent_ref[...]
    cols = jnp.where(pl.program_id(1) == 1, blk[:, EMB:], blk[:, :EMB])
    out_ref[...] = lax.dot_general(
        cols, u_s[...], (((1,), (1,)), ((), ())),
        preferred_element_type=jnp.float32)


def _final(g, wu, bu, ent_fold):
    outT = pl.pallas_call(
        _k4_body, grid=(NB, 2),
        in_specs=[pl.BlockSpec((N_IDX, EMB), lambda j, h: (0, 0)),
                  pl.BlockSpec((3 * EMB, EMB), lambda j, h: (0, 0)),
                  pl.BlockSpec((1, EMB), lambda j, h: (0, 0)),
                  pl.BlockSpec((FB, 2 * HIDDEN), lambda j, h: (j, 0))],
        out_specs=pl.BlockSpec((FB, BATCH), lambda j, h: (NB * h + j, 0)),
        out_shape=_f32(N_NODES, BATCH),
        scratch_shapes=[pltpu.VMEM((BATCH, EMB), jnp.float32)],
    )(g, wu, bu, ent_fold)
    return outT.T


def _pad_edges(ei):
    pad = E_PAD - E_PER
    src = ei[0]
    pre = src * 8 - jnp.where(src >= HALF, 8 * HALF - 4, 0)
    pre = jnp.concatenate([pre, jnp.zeros((pad,), jnp.int32)])
    pres = [(pre + q).reshape(NS, N_CH, CHUNK) for q in range(NQ)]
    dpad = N_NODES + (jnp.arange(pad, dtype=jnp.int32) % 16)
    dst = jnp.concatenate([ei[1], dpad]).reshape(NS, N_CH, CHUNK)
    return pres, dst


def kernel(edge_index_0, edge_index_1, edge_index_2,
           liked_indices, unknown_indices, disliked_indices,
           embed,
           W1_0, b1_0, W1_1, b1_1, W1_2, b1_2,
           W2_0, b2_0, W2_1, b2_1, W2_2, b2_2,
           Wu, bu):
    edges = [_pad_edges(e) for e in (edge_index_0, edge_index_1, edge_index_2)]
    b1s = [b.reshape(1, HIDDEN) for b in (b1_0, b1_1, b1_2)]
    b2s = [b.reshape(1, EMB) for b in (b2_0, b2_1, b2_2)]
    zrows = jnp.zeros((STG, QW), jnp.float32)
    zcnt = jnp.zeros((TPT,), jnp.float32)

    def qview(t):
        return t.reshape(8 * HALF, QW)

    pres = [p for e in range(3) for p in edges[e][0]]
    dsts = [edges[e][1] for e in range(3)]

    t1 = _linear1(embed, (W1_0, W1_1, W1_2), b1s)
    agg1 = _make_agg(True)
    s1_0, s1_1, s1_2, cnt = agg1(
        *pres, *dsts, *[qview(t) for t in t1], zrows, zcnt)
    cnt = cnt.reshape(3, NPAD)

    t2 = _mid((s1_0, s1_1, s1_2), cnt, (W2_0, W2_1, W2_2), b2s)
    agg2 = _make_agg(False)
    s2_0, s2_1, s2_2 = agg2(
        *pres, *dsts, *[qview(t) for t in t2], zrows, zcnt)

    ent_fold = _entities((s2_0, s2_1, s2_2), cnt)

    idxs = jnp.concatenate([liked_indices.reshape(-1),
                            disliked_indices.reshape(-1),
                            unknown_indices.reshape(-1)])
    idxs = idxs * 2 - jnp.where(idxs >= HALF, 2 * HALF - 1, 0)
    g = _gather_hist(ent_fold.reshape(2 * HALF, EMB), idxs)
    return _final(g, Wu, bu.reshape(1, EMB), ent_fold)
